# baseline (speedup 1.0000x reference)
import functools

import jax
import jax.numpy as jnp
from jax.experimental import pallas as pl
from jax.experimental.pallas import tpu as pltpu

K = 25
PAD = (K - 1) // 2
NE = 8
B, L, P = 1024, 1024, 1024
HID = 64
UW = 0.3


def _moe_kernel(z_ref, cov_ref, wear_ref, rw1_ref, rb1_ref, rw2_ref, rb2_ref,
                bs_ref, bt_ref, b1_ref, b2_ref,
                ws_ref, wt_ref, w1_ref, w2_ref,
                out_ref, res_ref, mm_ref, wvec_ref):
    i = pl.program_id(0)
    dn = (((1,), (1,)), ((), ()))
    f32 = jnp.float32

    @pl.when(i == 0)
    def _prep():
        Z = z_ref[...]
        jc = jax.lax.broadcasted_iota(jnp.int32, (L, L), 0).astype(f32)
        lc = jax.lax.broadcasted_iota(jnp.int32, (L, L), 1).astype(f32)
        band = (jnp.abs(jc - lc) <= PAD).astype(f32)
        front = jnp.where(lc == 0, jnp.maximum(PAD - jc, 0.0), 0.0)
        back = jnp.where(lc == L - 1, jnp.maximum(jc - (L - 1 - PAD), 0.0), 0.0)
        A = (band + front + back) * (1.0 / K)
        mm = jax.lax.dot_general(Z, A, dn, preferred_element_type=f32)
        mm_ref[...] = mm
        res_ref[...] = Z - mm
        comb = jnp.concatenate([cov_ref[...], wear_ref[...]], axis=1)
        hr = jnp.maximum(
            jax.lax.dot_general(comb, rw1_ref[...], dn,
                                preferred_element_type=f32) + rb1_ref[...],
            0.0)
        logits = jax.lax.dot_general(hr, rw2_ref[...], dn,
                                     preferred_element_type=f32) + rb2_ref[...]
        mx = jnp.max(logits, axis=1, keepdims=True)
        e = jnp.exp(logits - mx)
        sm = e / jnp.sum(e, axis=1, keepdims=True) * (1.0 - UW)
        wvec_ref[...] = jnp.concatenate(
            [jnp.full((B, 1), UW, f32), sm], axis=1)
        out_ref[...] = jax.lax.dot_general(
            wvec_ref[...], b2_ref[...], (((1,), (0,)), ((), ())),
            preferred_element_type=f32)

    dec = (jax.lax.dot_general(res_ref[...], ws_ref[0], dn,
                               preferred_element_type=f32)
           + jax.lax.dot_general(mm_ref[...], wt_ref[0], dn,
                                 preferred_element_type=f32)
           + bs_ref[0] + bt_ref[0])
    h = jnp.maximum(
        jax.lax.dot_general(dec, w1_ref[0], dn, preferred_element_type=f32)
        + b1_ref[0], 0.0)
    lane = jax.lax.broadcasted_iota(jnp.int32, (1, NE), 1)
    w = jnp.sum(wvec_ref[...] * (lane == i).astype(f32), axis=1, keepdims=True)
    g = w * h
    o = jax.lax.dot_general(g, w2_ref[0], dn, preferred_element_type=f32)
    out_ref[...] += o


@functools.partial(jax.jit, static_argnames=())
def kernel(z, cov_embedding, wearable_embedding, expert_Ws, expert_bs,
           expert_Wt, expert_bt, expert_W1, expert_b1, expert_W2, expert_b2,
           router_W1, router_b1, router_W2, router_b2):
    zsq = z[:, :, 0]
    rb1 = router_b1.reshape(1, HID)
    rb2 = router_b2.reshape(1, NE - 1)
    bsr = expert_bs.reshape(NE, 1, P)
    btr = expert_bt.reshape(NE, 1, P)
    b1r = expert_b1.reshape(NE, 1, HID)

    out = pl.pallas_call(
        _moe_kernel,
        grid=(NE,),
        in_specs=[
            pl.BlockSpec((B, L), lambda i: (0, 0)),
            pl.BlockSpec((B, HID), lambda i: (0, 0)),
            pl.BlockSpec((B, HID), lambda i: (0, 0)),
            pl.BlockSpec((HID, 2 * HID), lambda i: (0, 0)),
            pl.BlockSpec((1, HID), lambda i: (0, 0)),
            pl.BlockSpec((NE - 1, HID), lambda i: (0, 0)),
            pl.BlockSpec((1, NE - 1), lambda i: (0, 0)),
            pl.BlockSpec((1, 1, P), lambda i: (i, 0, 0)),
            pl.BlockSpec((1, 1, P), lambda i: (i, 0, 0)),
            pl.BlockSpec((1, 1, HID), lambda i: (i, 0, 0)),
            pl.BlockSpec((NE, P), lambda i: (0, 0)),
            pl.BlockSpec((1, P, L), lambda i: (i, 0, 0)),
            pl.BlockSpec((1, P, L), lambda i: (i, 0, 0)),
            pl.BlockSpec((1, HID, P), lambda i: (i, 0, 0)),
            pl.BlockSpec((1, P, HID), lambda i: (i, 0, 0)),
        ],
        out_specs=pl.BlockSpec((B, P), lambda i: (0, 0)),
        out_shape=jax.ShapeDtypeStruct((B, P), jnp.float32),
        compiler_params=pltpu.CompilerParams(
            dimension_semantics=("arbitrary",),
            vmem_limit_bytes=100 * 1024 * 1024,
        ),
        scratch_shapes=[
            pltpu.VMEM((B, L), jnp.float32),
            pltpu.VMEM((B, L), jnp.float32),
            pltpu.VMEM((B, NE), jnp.float32),
        ],
    )(zsq, cov_embedding, wearable_embedding, router_W1, rb1, router_W2, rb2,
      bsr, btr, b1r, expert_b2,
      expert_Ws, expert_Wt, expert_W1, expert_W2)

    return out[..., None]


# speedup vs baseline: 1.0516x; 1.0516x over previous
import functools

import jax
import jax.numpy as jnp
from jax.experimental import pallas as pl
from jax.experimental.pallas import tpu as pltpu

K = 25
PAD = (K - 1) // 2
NE = 8
B, L, P = 1024, 1024, 1024
HP = P // 2
HID = 64
UW = 0.3


def _moe_kernel(z_ref, cov_ref, wear_ref, rw1_ref, rb1_ref, rw2_ref, rb2_ref,
                bs_ref, bt_ref, b1_ref, b2_ref,
                wsa_ref, wsb_ref, wta_ref, wtb_ref, w1_ref, w2_ref,
                out_ref, res_ref, mm_ref, wvec_ref, dta_ref, dtb_ref):
    i = pl.program_id(0)
    dn = (((1,), (1,)), ((), ()))
    f32 = jnp.float32

    @pl.when(i == 0)
    def _prep():
        Z = z_ref[...]
        jc = jax.lax.broadcasted_iota(jnp.int32, (L, L), 0).astype(f32)
        lc = jax.lax.broadcasted_iota(jnp.int32, (L, L), 1).astype(f32)
        band = (jnp.abs(jc - lc) <= PAD).astype(f32)
        front = jnp.where(lc == 0, jnp.maximum(PAD - jc, 0.0), 0.0)
        back = jnp.where(lc == L - 1, jnp.maximum(jc - (L - 1 - PAD), 0.0), 0.0)
        A = (band + front + back) * (1.0 / K)
        mm = jax.lax.dot_general(Z, A, dn, preferred_element_type=f32)
        mm_ref[...] = mm
        res_ref[...] = Z - mm
        comb = jnp.concatenate([cov_ref[...], wear_ref[...]], axis=1)
        hr = jnp.maximum(
            jax.lax.dot_general(comb, rw1_ref[...], dn,
                                preferred_element_type=f32) + rb1_ref[...],
            0.0)
        logits = jax.lax.dot_general(hr, rw2_ref[...], dn,
                                     preferred_element_type=f32) + rb2_ref[...]
        mx = jnp.max(logits, axis=1, keepdims=True)
        e = jnp.exp(logits - mx)
        sm = e / jnp.sum(e, axis=1, keepdims=True) * (1.0 - UW)
        wvec_ref[...] = jnp.concatenate(
            [jnp.full((B, 1), UW, f32), sm], axis=1)
        dta_ref[1] = jnp.zeros((B, HP), f32)
        dtb_ref[1] = jnp.zeros((B, HP), f32)
        out_ref[...] = jax.lax.dot_general(
            wvec_ref[...], b2_ref[...], (((1,), (0,)), ((), ())),
            preferred_element_type=f32)

    parity = jax.lax.rem(i, 2)

    @pl.when(i < NE)
    def _decode():
        res = res_ref[...]
        mm = mm_ref[...]
        bias = bs_ref[0] + bt_ref[0]
        da = (jax.lax.dot_general(res, wsa_ref[0], dn,
                                  preferred_element_type=f32)
              + jax.lax.dot_general(mm, wta_ref[0], dn,
                                    preferred_element_type=f32)
              + bias[:, :HP])
        db = (jax.lax.dot_general(res, wsb_ref[0], dn,
                                  preferred_element_type=f32)
              + jax.lax.dot_general(mm, wtb_ref[0], dn,
                                    preferred_element_type=f32)
              + bias[:, HP:])
        dta_ref[parity] = da
        dtb_ref[parity] = db

    dA = dta_ref[1 - parity]
    dB = dtb_ref[1 - parity]
    w1 = w1_ref[0]
    h = jnp.maximum(
        jax.lax.dot_general(dA, w1[:, :HP], dn, preferred_element_type=f32)
        + jax.lax.dot_general(dB, w1[:, HP:], dn, preferred_element_type=f32)
        + b1_ref[0], 0.0)
    lane = jax.lax.broadcasted_iota(jnp.int32, (1, NE), 1)
    w = jnp.sum(wvec_ref[...] * (lane == (i - 1)).astype(f32),
                axis=1, keepdims=True)
    g = w * h
    w2 = w2_ref[0]
    oa = jax.lax.dot_general(g, w2[:HP, :], dn, preferred_element_type=f32)
    ob = jax.lax.dot_general(g, w2[HP:, :], dn, preferred_element_type=f32)
    out_ref[:, :HP] += oa
    out_ref[:, HP:] += ob


@functools.partial(jax.jit, static_argnames=())
def kernel(z, cov_embedding, wearable_embedding, expert_Ws, expert_bs,
           expert_Wt, expert_bt, expert_W1, expert_b1, expert_W2, expert_b2,
           router_W1, router_b1, router_W2, router_b2):
    zsq = z[:, :, 0]
    rb1 = router_b1.reshape(1, HID)
    rb2 = router_b2.reshape(1, NE - 1)
    bsr = expert_bs.reshape(NE, 1, P)
    btr = expert_bt.reshape(NE, 1, P)
    b1r = expert_b1.reshape(NE, 1, HID)

    def dec_idx_a(i):
        return (jnp.minimum(i, NE - 1), 0, 0)

    def dec_idx_b(i):
        return (jnp.minimum(i, NE - 1), 1, 0)

    def mlp_idx(i):
        return (jnp.maximum(i - 1, 0), 0, 0)

    out = pl.pallas_call(
        _moe_kernel,
        grid=(NE + 1,),
        in_specs=[
            pl.BlockSpec((B, L), lambda i: (0, 0)),
            pl.BlockSpec((B, HID), lambda i: (0, 0)),
            pl.BlockSpec((B, HID), lambda i: (0, 0)),
            pl.BlockSpec((HID, 2 * HID), lambda i: (0, 0)),
            pl.BlockSpec((1, HID), lambda i: (0, 0)),
            pl.BlockSpec((NE - 1, HID), lambda i: (0, 0)),
            pl.BlockSpec((1, NE - 1), lambda i: (0, 0)),
            pl.BlockSpec((1, 1, P), dec_idx_a),
            pl.BlockSpec((1, 1, P), dec_idx_a),
            pl.BlockSpec((1, 1, HID), mlp_idx),
            pl.BlockSpec((NE, P), lambda i: (0, 0)),
            pl.BlockSpec((1, HP, L), dec_idx_a),
            pl.BlockSpec((1, HP, L), dec_idx_b),
            pl.BlockSpec((1, HP, L), dec_idx_a),
            pl.BlockSpec((1, HP, L), dec_idx_b),
            pl.BlockSpec((1, HID, P), mlp_idx),
            pl.BlockSpec((1, P, HID), mlp_idx),
        ],
        out_specs=pl.BlockSpec((B, P), lambda i: (0, 0)),
        out_shape=jax.ShapeDtypeStruct((B, P), jnp.float32),
        compiler_params=pltpu.CompilerParams(
            dimension_semantics=("arbitrary",),
            vmem_limit_bytes=100 * 1024 * 1024,
        ),
        scratch_shapes=[
            pltpu.VMEM((B, L), jnp.float32),
            pltpu.VMEM((B, L), jnp.float32),
            pltpu.VMEM((B, NE), jnp.float32),
            pltpu.VMEM((2, B, HP), jnp.float32),
            pltpu.VMEM((2, B, HP), jnp.float32),
        ],
    )(zsq, cov_embedding, wearable_embedding, router_W1, rb1, router_W2, rb2,
      bsr, btr, b1r, expert_b2,
      expert_Ws, expert_Ws, expert_Wt, expert_Wt, expert_W1, expert_W2)

    return out[..., None]
